# Initial kernel scaffold; baseline (speedup 1.0000x reference)
import jax
import jax.numpy as jnp
from jax.experimental import pallas as pl


def kernel(features, edge_index, edgenet_input, batch, conv0_W0, conv0_W1, conv1_W0, conv1_W1, conv2_W0, conv2_W1, bn0_g, bn0_b, bn1_g, bn1_b, bn2_g, bn2_b, cls_W1, cls_b1, cls_bn_g, cls_bn_b, cls_W2, cls_b2):
    raise NotImplementedError("write your pallas kernel here")


# speedup vs baseline: 21.5123x; 21.5123x over previous
import functools

import jax
import jax.numpy as jnp
from jax import lax
from jax.experimental import pallas as pl
from jax.experimental.pallas import tpu as pltpu
from jax.experimental.pallas import tpu_sc as plsc

N = 10000
E = 320000
D_IN = 128
HGC = 64
NG = 512
JK = 192
BN_EPS = 1e-5
ISQ = float(1.0 / (1.0 + BN_EPS) ** 0.5)

NC, NS = 2, 16
NW = NC * NS
EPC = E // NC
EPW = EPC // NS
ECH = 1000
NCHUNK = EPW // ECH
NPAD = 10240
DPT = NPAD // NS
NPW = 312
NREM = N - NPW * NW
RPT = 624
RREM = N - RPT * NS
PPT = NG // NS

_sc_mesh = plsc.VectorSubcoreMesh(core_axis_name="c", subcore_axis_name="s")


@functools.partial(
    pl.kernel,
    out_type=(jax.ShapeDtypeStruct((NC, NPAD), jnp.float32),
              jax.ShapeDtypeStruct((NC, NG), jnp.float32)),
    mesh=_sc_mesh,
    compiler_params=pltpu.CompilerParams(use_tc_tiling_on_sc=False),
    scratch_types=[
        pltpu.VMEM((ECH,), jnp.int32),
        pltpu.VMEM((NPW,), jnp.int32),
        pltpu.VMEM((NREM,), jnp.int32),
        pltpu.VMEM((ECH,), jnp.float32),
        pltpu.VMEM((DPT,), jnp.float32),
        pltpu.VMEM_SHARED((NPAD,), jnp.float32),
        pltpu.VMEM_SHARED((NG,), jnp.float32),
    ],
)
def _sc_degcnt(row_hbm, batch_hbm, ones_hbm, zeros_hbm, deg_out, cnt_out,
               rid_v, bid_v, bid16_v, ones_v, zb_v, dacc, cacc):
    c = lax.axis_index("c")
    s = lax.axis_index("s")
    w = c * NS + s
    pltpu.sync_copy(zeros_hbm.at[pl.ds(0, DPT)], zb_v)
    pltpu.sync_copy(zb_v, dacc.at[pl.ds(s * DPT, DPT)])
    pltpu.sync_copy(zb_v.at[pl.ds(0, PPT)], cacc.at[pl.ds(s * PPT, PPT)])
    pltpu.sync_copy(ones_hbm, ones_v)
    plsc.subcore_barrier()

    def body(k, carry):
        base = c * EPC + s * EPW + k * ECH
        pltpu.sync_copy(row_hbm.at[pl.ds(base, ECH)], rid_v)
        pltpu.sync_copy(ones_v, dacc.at[rid_v], add=True)
        return carry
    lax.fori_loop(0, NCHUNK, body, 0)

    pltpu.sync_copy(batch_hbm.at[pl.ds(w * NPW, NPW)], bid_v)
    pltpu.sync_copy(ones_v.at[pl.ds(0, NPW)], cacc.at[bid_v], add=True)

    @pl.when(w == NW - 1)
    def _():
        pltpu.sync_copy(batch_hbm.at[pl.ds(NW * NPW, NREM)], bid16_v)
        pltpu.sync_copy(ones_v.at[pl.ds(0, NREM)], cacc.at[bid16_v], add=True)

    plsc.subcore_barrier()
    pltpu.sync_copy(dacc.at[pl.ds(s * DPT, DPT)], zb_v)
    pltpu.sync_copy(zb_v, deg_out.at[c, pl.ds(s * DPT, DPT)])
    pltpu.sync_copy(cacc.at[pl.ds(s * PPT, PPT)], zb_v.at[pl.ds(0, PPT)])
    pltpu.sync_copy(zb_v.at[pl.ds(0, PPT)], cnt_out.at[c, pl.ds(s * PPT, PPT)])


@functools.partial(
    pl.kernel,
    out_type=jax.ShapeDtypeStruct((NC, N, HGC), jnp.float32),
    mesh=_sc_mesh,
    compiler_params=pltpu.CompilerParams(use_tc_tiling_on_sc=False),
    scratch_types=[
        pltpu.VMEM((ECH,), jnp.int32),
        pltpu.VMEM((ECH,), jnp.int32),
        pltpu.VMEM((ECH, HGC), jnp.float32),
        pltpu.VMEM_SHARED((N, HGC), jnp.float32),
    ],
)
def _sc_prop(ys_hbm, row_hbm, col_hbm, zeros_hbm, out_hbm,
             rid_v, cid_v, rows_v, acc):
    c = lax.axis_index("c")
    s = lax.axis_index("s")
    pltpu.sync_copy(zeros_hbm.at[pl.ds(s * RPT, RPT)], rows_v.at[pl.ds(0, RPT)])
    pltpu.sync_copy(rows_v.at[pl.ds(0, RPT)], acc.at[pl.ds(s * RPT, RPT)])

    @pl.when(s == NS - 1)
    def _():
        pltpu.sync_copy(zeros_hbm.at[pl.ds(NS * RPT, RREM)],
                        rows_v.at[pl.ds(RPT, RREM)])
        pltpu.sync_copy(rows_v.at[pl.ds(RPT, RREM)], acc.at[pl.ds(NS * RPT, RREM)])

    plsc.subcore_barrier()

    def body(k, carry):
        base = c * EPC + s * EPW + k * ECH
        pltpu.sync_copy(row_hbm.at[pl.ds(base, ECH)], rid_v)
        pltpu.sync_copy(col_hbm.at[pl.ds(base, ECH)], cid_v)
        pltpu.sync_copy(ys_hbm.at[rid_v], rows_v)
        pltpu.sync_copy(rows_v, acc.at[cid_v], add=True)
        return carry
    lax.fori_loop(0, NCHUNK, body, 0)

    plsc.subcore_barrier()
    pltpu.sync_copy(acc.at[pl.ds(s * RPT, RPT)], rows_v.at[pl.ds(0, RPT)])
    pltpu.sync_copy(rows_v.at[pl.ds(0, RPT)], out_hbm.at[c, pl.ds(s * RPT, RPT)])

    @pl.when(s == NS - 1)
    def _():
        pltpu.sync_copy(acc.at[pl.ds(NS * RPT, RREM)], rows_v.at[pl.ds(RPT, RREM)])
        pltpu.sync_copy(rows_v.at[pl.ds(RPT, RREM)],
                        out_hbm.at[c, pl.ds(NS * RPT, RREM)])


@functools.partial(
    pl.kernel,
    out_type=jax.ShapeDtypeStruct((NC, NG, JK), jnp.float32),
    mesh=_sc_mesh,
    compiler_params=pltpu.CompilerParams(use_tc_tiling_on_sc=False),
    scratch_types=[
        pltpu.VMEM((NPW,), jnp.int32),
        pltpu.VMEM((NREM,), jnp.int32),
        pltpu.VMEM((NPW, JK), jnp.float32),
        pltpu.VMEM((NREM, JK), jnp.float32),
        pltpu.VMEM_SHARED((NG, JK), jnp.float32),
    ],
)
def _sc_pool(jk_hbm, batch_hbm, zeros_hbm, out_hbm,
             bid_v, bid16_v, rows_v, rows16_v, acc):
    c = lax.axis_index("c")
    s = lax.axis_index("s")
    w = c * NS + s
    pltpu.sync_copy(zeros_hbm.at[pl.ds(s * PPT, PPT)], rows_v.at[pl.ds(0, PPT)])
    pltpu.sync_copy(rows_v.at[pl.ds(0, PPT)], acc.at[pl.ds(s * PPT, PPT)])
    plsc.subcore_barrier()

    pltpu.sync_copy(batch_hbm.at[pl.ds(w * NPW, NPW)], bid_v)
    pltpu.sync_copy(jk_hbm.at[pl.ds(w * NPW, NPW)], rows_v)
    pltpu.sync_copy(rows_v, acc.at[bid_v], add=True)

    @pl.when(w == NW - 1)
    def _():
        pltpu.sync_copy(batch_hbm.at[pl.ds(NW * NPW, NREM)], bid16_v)
        pltpu.sync_copy(jk_hbm.at[pl.ds(NW * NPW, NREM)], rows16_v)
        pltpu.sync_copy(rows16_v, acc.at[bid16_v], add=True)

    plsc.subcore_barrier()
    pltpu.sync_copy(acc.at[pl.ds(s * PPT, PPT)], rows_v.at[pl.ds(0, PPT)])
    pltpu.sync_copy(rows_v.at[pl.ds(0, PPT)], out_hbm.at[c, pl.ds(s * PPT, PPT)])


RB = 2000
GRID = N // RB


def _tc0(features, degp_col, wcat0):
    def body(f_ref, d_ref, w_ref, dis_ref, xw0_ref, ys_ref):
        deg = d_ref[0] + d_ref[1]
        dis = jnp.where(deg > 0, lax.rsqrt(jnp.maximum(deg, 1.0)), 0.0)
        cat = jnp.dot(f_ref[...], w_ref[...], preferred_element_type=jnp.float32)
        dis_ref[...] = dis
        xw0_ref[...] = cat[:, :HGC]
        ys_ref[...] = dis * cat[:, HGC:]
    return pl.pallas_call(
        body,
        grid=(GRID,),
        in_specs=[pl.BlockSpec((RB, D_IN), lambda i: (i, 0)),
                  pl.BlockSpec((NC, RB, 1), lambda i: (0, i, 0)),
                  pl.BlockSpec((D_IN, 2 * HGC), lambda i: (0, 0))],
        out_specs=[pl.BlockSpec((RB, 1), lambda i: (i, 0)),
                   pl.BlockSpec((RB, HGC), lambda i: (i, 0)),
                   pl.BlockSpec((RB, HGC), lambda i: (i, 0))],
        out_shape=[jax.ShapeDtypeStruct((N, 1), jnp.float32),
                   jax.ShapeDtypeStruct((N, HGC), jnp.float32),
                   jax.ShapeDtypeStruct((N, HGC), jnp.float32)],
    )(features, degp_col, wcat0)


def _tc_mid(xw0, pp, dis, g, b, wcat, bn_first):
    def body(x_ref, p_ref, d_ref, g_ref, b_ref, w_ref, h_ref, xw0_ref, ys_ref):
        dis = d_ref[...]
        t = x_ref[...] - dis * (p_ref[0] + p_ref[1])
        if bn_first:
            h = jax.nn.relu(g_ref[...] * ISQ * t + b_ref[...])
        else:
            h = g_ref[...] * ISQ * jax.nn.relu(t) + b_ref[...]
        cat = jnp.dot(h, w_ref[...], preferred_element_type=jnp.float32)
        h_ref[...] = h
        xw0_ref[...] = cat[:, :HGC]
        ys_ref[...] = dis * cat[:, HGC:]
    return pl.pallas_call(
        body,
        grid=(GRID,),
        in_specs=[pl.BlockSpec((RB, HGC), lambda i: (i, 0)),
                  pl.BlockSpec((NC, RB, HGC), lambda i: (0, i, 0)),
                  pl.BlockSpec((RB, 1), lambda i: (i, 0)),
                  pl.BlockSpec((1, HGC), lambda i: (0, 0)),
                  pl.BlockSpec((1, HGC), lambda i: (0, 0)),
                  pl.BlockSpec((HGC, 2 * HGC), lambda i: (0, 0))],
        out_specs=[pl.BlockSpec((RB, HGC), lambda i: (i, 0)),
                   pl.BlockSpec((RB, HGC), lambda i: (i, 0)),
                   pl.BlockSpec((RB, HGC), lambda i: (i, 0))],
        out_shape=[jax.ShapeDtypeStruct((N, HGC), jnp.float32),
                   jax.ShapeDtypeStruct((N, HGC), jnp.float32),
                   jax.ShapeDtypeStruct((N, HGC), jnp.float32)],
    )(xw0, pp, dis, g, b, wcat)


def _tc_last(xw0, pp, dis, g, b, h0, h1):
    def body(x_ref, p_ref, d_ref, g_ref, b_ref, h0_ref, h1_ref, jk_ref):
        t = x_ref[...] - d_ref[...] * (p_ref[0] + p_ref[1])
        h2 = g_ref[...] * ISQ * jax.nn.relu(t) + b_ref[...]
        jk_ref[...] = jnp.concatenate([h0_ref[...], h1_ref[...], h2], axis=1)
    return pl.pallas_call(
        body,
        grid=(GRID,),
        in_specs=[pl.BlockSpec((RB, HGC), lambda i: (i, 0)),
                  pl.BlockSpec((NC, RB, HGC), lambda i: (0, i, 0)),
                  pl.BlockSpec((RB, 1), lambda i: (i, 0)),
                  pl.BlockSpec((1, HGC), lambda i: (0, 0)),
                  pl.BlockSpec((1, HGC), lambda i: (0, 0)),
                  pl.BlockSpec((RB, HGC), lambda i: (i, 0)),
                  pl.BlockSpec((RB, HGC), lambda i: (i, 0))],
        out_specs=pl.BlockSpec((RB, JK), lambda i: (i, 0)),
        out_shape=jax.ShapeDtypeStruct((N, JK), jnp.float32),
    )(xw0, pp, dis, g, b, h0, h1)


def _tc_head(sums_p, cnt_col, w1, b1, g, b, w2, b2):
    def body(s_ref, c_ref, w1_ref, b1_ref, g_ref, b_ref, w2_ref, b2_ref,
             zagg_ref, logit_ref):
        sums = s_ref[0] + s_ref[1]
        cnt = c_ref[0] + c_ref[1]
        z_agg = sums / jnp.maximum(cnt, 1.0)
        z = jax.nn.relu(jnp.dot(z_agg, w1_ref[...],
                                preferred_element_type=jnp.float32) + b1_ref[...])
        z = g_ref[...] * ISQ * z + b_ref[...]
        logit = jnp.dot(z, w2_ref[...],
                        preferred_element_type=jnp.float32) + b2_ref[...]
        zagg_ref[...] = z_agg
        logit_ref[...] = logit
    return pl.pallas_call(
        body,
        out_shape=[jax.ShapeDtypeStruct((NG, JK), jnp.float32),
                   jax.ShapeDtypeStruct((NG, 2), jnp.float32)],
    )(sums_p, cnt_col, w1, b1, g, b, w2, b2)


def kernel(features, edge_index, edgenet_input, batch,
           conv0_W0, conv0_W1, conv1_W0, conv1_W1, conv2_W0, conv2_W1,
           bn0_g, bn0_b, bn1_g, bn1_b, bn2_g, bn2_b,
           cls_W1, cls_b1, cls_bn_g, cls_bn_b, cls_W2, cls_b2):
    row = edge_index[0]
    col = edge_index[1]
    ones1k = jnp.ones((ECH,), jnp.float32)
    zflat = jnp.zeros((NPAD,), jnp.float32)
    zn64 = jnp.zeros((N, HGC), jnp.float32)
    zpool = jnp.zeros((NG, JK), jnp.float32)
    wcat0 = jnp.concatenate([conv0_W0, conv0_W1], axis=1)
    wcat1 = jnp.concatenate([conv1_W0, conv1_W1], axis=1)
    wcat2 = jnp.concatenate([conv2_W0, conv2_W1], axis=1)

    degp, cntp = _sc_degcnt(row, batch, ones1k, zflat)
    degp_col = degp.reshape(NC, NPAD, 1)[:, :N]

    dis, xw0_0, ys0 = _tc0(features, degp_col, wcat0)
    pp0 = _sc_prop(ys0, row, col, zn64)
    h0, xw0_1, ys1 = _tc_mid(xw0_0, pp0, dis, bn0_g.reshape(1, HGC),
                             bn0_b.reshape(1, HGC), wcat1, bn_first=True)
    pp1 = _sc_prop(ys1, row, col, zn64)
    h1, xw0_2, ys2 = _tc_mid(xw0_1, pp1, dis, bn1_g.reshape(1, HGC),
                             bn1_b.reshape(1, HGC), wcat2, bn_first=False)
    pp2 = _sc_prop(ys2, row, col, zn64)
    jk = _tc_last(xw0_2, pp2, dis, bn2_g.reshape(1, HGC),
                  bn2_b.reshape(1, HGC), h0, h1)

    sp = _sc_pool(jk, batch, zpool)
    z_agg, logit = _tc_head(sp, cntp.reshape(NC, NG, 1),
                            cls_W1, cls_b1.reshape(1, 256),
                            cls_bn_g.reshape(1, 256), cls_bn_b.reshape(1, 256),
                            cls_W2, cls_b2.reshape(1, 2))
    return (z_agg, logit)
